# baseline (speedup 1.0000x reference)
import functools

import jax
import jax.numpy as jnp
from jax import lax
from jax.experimental import pallas as pl
from jax.experimental.pallas import tpu as pltpu
from jax.experimental.pallas import tpu_sc as plsc

_NC = 2
_NS = 16
_NW = _NC * _NS
_SPLIT = 4
_NBUF = 8


def _sc_gather_body(x4, gall, sel4, rem4, idx_v, *rest, pieces_pw):
    bufs = rest[:_NBUF]
    gsems = rest[_NBUF:2 * _NBUF]
    osems = rest[2 * _NBUF:3 * _NBUF]
    wid = lax.axis_index("s") * _NC + lax.axis_index("c")
    pltpu.sync_copy(gall.at[pl.ds(wid * pieces_pw, pieces_pw)], idx_v)
    half = wid // _NS
    obase = (wid % _NS) * pieces_pw

    def run(out4):
        for j in range(_NBUF):
            pltpu.async_copy(x4.at[idx_v.at[j]], bufs[j], gsems[j])
        for k in range(pieces_pw):
            b = k % _NBUF
            pltpu.make_async_copy(
                x4.at[idx_v.at[k]], bufs[b], gsems[b]).wait()
            dst = out4.at[pl.ds(obase + k, 1)]
            pltpu.async_copy(bufs[b], dst, osems[b])
            if k + _NBUF < pieces_pw:
                pltpu.make_async_copy(bufs[b], dst, osems[b]).wait()
                pltpu.async_copy(
                    x4.at[idx_v.at[k + _NBUF]], bufs[b], gsems[b])
        for k in range(pieces_pw - _NBUF, pieces_pw):
            b = k % _NBUF
            pltpu.make_async_copy(
                bufs[b], out4.at[pl.ds(obase + k, 1)], osems[b]).wait()

    @pl.when(half == 0)
    def _():
        run(sel4)

    @pl.when(half == 1)
    def _():
        run(rem4)


def kernel(x, W1, b1, W2, b2):
    N, C, H, W = x.shape
    rc = C // 2
    row_el = H * W
    piece_el = row_el // _SPLIT
    pieces_pw = (N * C * _SPLIT) // _NW
    z = jnp.mean(x, axis=(2, 3))
    s = jax.nn.relu(z @ W1.T + b1)
    s = jax.nn.sigmoid(s @ W2.T + b2)
    indices = jnp.argsort(-s, axis=1).astype(jnp.int32)

    rows = jnp.arange(N, dtype=jnp.int32)[:, None] * C + indices
    gall = jnp.concatenate(
        [rows[:, :rc].reshape(-1), rows[:, rc:].reshape(-1)])
    gall = (gall[:, None] * _SPLIT
            + jnp.arange(_SPLIT, dtype=jnp.int32)[None, :]).reshape(-1, 1)

    x4 = x.reshape(N * C * _SPLIT, piece_el)
    body = functools.partial(_sc_gather_body, pieces_pw=pieces_pw)
    sel4, rem4 = pl.kernel(
        body,
        out_type=[
            jax.ShapeDtypeStruct((N * rc * _SPLIT, piece_el), x.dtype),
            jax.ShapeDtypeStruct((N * (C - rc) * _SPLIT, piece_el), x.dtype),
        ],
        mesh=plsc.VectorSubcoreMesh(core_axis_name="c", subcore_axis_name="s"),
        scratch_types=(
            [pltpu.VMEM((pieces_pw, 1), jnp.int32)]
            + [pltpu.VMEM((1, piece_el), jnp.float32)] * _NBUF
            + [pltpu.SemaphoreType.DMA] * (2 * _NBUF)
        ),
    )(x4, gall)
    sel = sel4.reshape(N, rc, H, W)
    rem = rem4.reshape(N, C - rc, H, W)
    return sel, rem


# speedup vs baseline: 3.2409x; 3.2409x over previous
import jax
import jax.numpy as jnp
from jax.experimental import pallas as pl
from jax.experimental.pallas import tpu as pltpu

_G = 24


def _gather_copy_kernel(idx_ref, *refs):
    xs = refs[:_G]
    xr = refs[_G:2 * _G]
    sel_ref, rem_ref = refs[2 * _G], refs[2 * _G + 1]
    for g in range(_G):
        sel_ref[0, g] = xs[g][0, 0]
        rem_ref[0, g] = xr[g][0, 0]


def kernel(x, W1, b1, W2, b2):
    N, C, H, W = x.shape
    rc = C // 2
    z = jnp.mean(x, axis=(2, 3))
    s = jax.nn.relu(z @ W1.T + b1)
    s = jax.nn.sigmoid(s @ W2.T + b2)
    indices = jnp.argsort(-s, axis=1).astype(jnp.int32)

    in_specs = [
        pl.BlockSpec((1, 1, H, W),
                     (lambda n, j, idx, g=g: (n, idx[n, j * _G + g], 0, 0)))
        for g in range(_G)
    ] + [
        pl.BlockSpec((1, 1, H, W),
                     (lambda n, j, idx, g=g: (n, idx[n, rc + j * _G + g], 0, 0)))
        for g in range(_G)
    ]
    grid_spec = pltpu.PrefetchScalarGridSpec(
        num_scalar_prefetch=1,
        grid=(N, rc // _G),
        in_specs=in_specs,
        out_specs=[
            pl.BlockSpec((1, _G, H, W), lambda n, j, idx: (n, j, 0, 0)),
            pl.BlockSpec((1, _G, H, W), lambda n, j, idx: (n, j, 0, 0)),
        ],
    )
    sel, rem = pl.pallas_call(
        _gather_copy_kernel,
        grid_spec=grid_spec,
        out_shape=[
            jax.ShapeDtypeStruct((N, rc, H, W), x.dtype),
            jax.ShapeDtypeStruct((N, C - rc, H, W), x.dtype),
        ],
    )(indices, *([x] * (2 * _G)))
    return sel, rem
